# baseline (speedup 1.0000x reference)
import functools

import jax
import jax.numpy as jnp
from jax import lax
from jax.experimental import pallas as pl
from jax.experimental.pallas import tpu as pltpu
from jax.experimental.pallas import tpu_sc as plsc

B = 4096
L = 200
VOCAB = 1000000
D = 64

NW = 32
TOTAL = B * L
ROWS128 = TOTAL // 128
ROWS_PER_W = ROWS128 // NW
CHUNK_ROWS = 5
NCHUNKS = ROWS_PER_W // CHUNK_ROWS
NBUF = 2

NBLK = VOCAB // 128
BLK_PER_W = NBLK // NW

_mesh = plsc.VectorSubcoreMesh(core_axis_name="c", subcore_axis_name="s")


@functools.partial(
    pl.kernel,
    mesh=_mesh,
    out_type=jax.ShapeDtypeStruct((VOCAB // 2, 128), jnp.float32),
    scratch_types=[
        pltpu.VMEM((D, 128), jnp.float32),
        pltpu.VMEM((D, 128), jnp.float32),
    ],
    compiler_params=pltpu.CompilerParams(use_tc_tiling_on_sc=True,
                                         needs_layout_passes=False),
)
def _transpose_kernel(tab_t, tail_in, out_hbm, src_v, dst_v):
    wid = lax.axis_index("s") * 2 + lax.axis_index("c")
    iota = lax.iota(jnp.int32, 16)

    def do_block(v0):
        v0 = pl.multiple_of(v0, 128)
        pltpu.sync_copy(tab_t.at[:, pl.ds(v0, 128)], src_v)

        def body(u, _):
            for half in range(2):
                vcol = 2 * u + half
                vidx = jnp.full((16,), vcol, jnp.int32)
                for d0 in range(4):
                    vals = plsc.load_gather(src_v, [d0 * 16 + iota, vidx])
                    dst_v[u, pl.ds(half * 64 + d0 * 16, 16)] = vals
            return 0

        lax.fori_loop(0, 64, body, 0)
        pltpu.sync_copy(dst_v,
                        out_hbm.at[pl.ds(pl.multiple_of(v0 // 2, 64), 64)])

    extra = jnp.where(wid < 4, 1, 0)
    start = wid * BLK_PER_W + jnp.minimum(wid, 4)
    count = BLK_PER_W + extra

    def blk_body(g, _):
        do_block((start + g) * 128)
        return 0

    lax.fori_loop(0, count, blk_body, 0)

    @pl.when(wid == NW - 1)
    def _():
        pltpu.sync_copy(tail_in, src_v.at[pl.ds(0, 32)])
        pltpu.sync_copy(src_v.at[pl.ds(0, 32)],
                        out_hbm.at[pl.ds(NBLK * 64, 32)])


@functools.partial(
    pl.kernel,
    mesh=_mesh,
    out_type=jax.ShapeDtypeStruct((ROWS128, 128, 128), jnp.float32),
    scratch_types=[
        pltpu.VMEM((NBUF, CHUNK_ROWS, 128), jnp.int32),
        pltpu.VMEM((NBUF, CHUNK_ROWS, 128, D), jnp.float32),
        pltpu.SemaphoreType.DMA,
        pltpu.SemaphoreType.DMA,
        pltpu.SemaphoreType.DMA,
        pltpu.SemaphoreType.DMA,
    ],
    compiler_params=pltpu.CompilerParams(use_tc_tiling_on_sc=False),
)
def _gather_kernel(tok_hbm, table_hbm, out_hbm, idx_v, rows_v,
                   gsem0, gsem1, ssem0, ssem1):
    wid = lax.axis_index("s") * 2 + lax.axis_index("c")
    base_row = wid * ROWS_PER_W
    gsem = (gsem0, gsem1)
    ssem = (ssem0, ssem1)

    def fire(g, b):
        row0 = base_row + g * CHUNK_ROWS
        pltpu.sync_copy(tok_hbm.at[pl.ds(row0, CHUNK_ROWS)], idx_v.at[b])
        for j in range(CHUNK_ROWS):
            pltpu.async_copy(table_hbm.at[idx_v.at[b, j]], rows_v.at[b, j],
                             gsem[b])

    def wait_gathers(b):
        pltpu.make_async_copy(
            out_hbm.at[pl.ds(0, CHUNK_ROWS), :, pl.ds(0, D)],
            rows_v.at[b], gsem[b]).wait()

    def fire_store(g, b):
        row0 = base_row + g * CHUNK_ROWS
        pltpu.async_copy(rows_v.at[b],
                         out_hbm.at[pl.ds(row0, CHUNK_ROWS), :, pl.ds(0, D)],
                         ssem[b])

    def wait_store(b):
        pltpu.make_async_copy(
            rows_v.at[b],
            out_hbm.at[pl.ds(0, CHUNK_ROWS), :, pl.ds(0, D)],
            ssem[b]).wait()

    fire(0, 0)
    fire(1, 1)
    wait_gathers(0)
    fire_store(0, 0)

    def body(k, _):
        g0 = 2 + 2 * k
        for b in range(NBUF):
            g = g0 + b
            wait_store(b)
            fire(g, b)
            wait_gathers(b ^ 1)
            fire_store(g - 1, b ^ 1)
        return 0

    lax.fori_loop(0, (NCHUNKS - 2) // 2, body, 0)

    b_last = (NCHUNKS - 1) % 2
    wait_gathers(b_last)
    fire_store(NCHUNKS - 1, b_last)
    wait_store(0)
    wait_store(1)


def kernel(tokens, word_embed_weight):
    tok = tokens.reshape(ROWS128, 128).astype(jnp.int32)
    tail_packed = word_embed_weight[NBLK * 128:].reshape(32, 128)
    packed = _transpose_kernel(word_embed_weight.T, tail_packed)
    table_rm = packed.reshape(VOCAB, D)
    out = _gather_kernel(tok, table_rm)
    return out.reshape(TOTAL, 128)[:, :D].reshape(B, L, D)


# speedup vs baseline: 1.1917x; 1.1917x over previous
import functools

import jax
import jax.numpy as jnp
from jax import lax
from jax.experimental import pallas as pl
from jax.experimental.pallas import tpu as pltpu
from jax.experimental.pallas import tpu_sc as plsc

B = 4096
L = 200
VOCAB = 1000000
D = 64

NW = 32
TOTAL = B * L
ROWS128 = TOTAL // 128
ROWS_PER_W = ROWS128 // NW
CHUNK_ROWS = 5
NCHUNKS = ROWS_PER_W // CHUNK_ROWS
NBUF = 2

TW = 256
NBLK = VOCAB // TW
BLK_PER_W = NBLK // NW
BLK_EXTRA = NBLK % NW

_mesh = plsc.VectorSubcoreMesh(core_axis_name="c", subcore_axis_name="s")


@functools.partial(
    pl.kernel,
    mesh=_mesh,
    out_type=jax.ShapeDtypeStruct((VOCAB // 2, 128), jnp.float32),
    scratch_types=[
        pltpu.VMEM((NBUF, D, TW), jnp.float32),
        pltpu.VMEM((NBUF, TW // 2, 128), jnp.float32),
        pltpu.SemaphoreType.DMA,
        pltpu.SemaphoreType.DMA,
        pltpu.SemaphoreType.DMA,
        pltpu.SemaphoreType.DMA,
    ],
    compiler_params=pltpu.CompilerParams(use_tc_tiling_on_sc=True,
                                         needs_layout_passes=False),
)
def _transpose_kernel(tab_t, tail_in, out_hbm, src_v, dst_v,
                      lsem0, lsem1, ssem0, ssem1):
    wid = lax.axis_index("s") * 2 + lax.axis_index("c")
    iota = lax.iota(jnp.int32, 16)
    lsem = (lsem0, lsem1)
    ssem = (ssem0, ssem1)

    extra = jnp.where(wid < BLK_EXTRA, 1, 0)
    start = wid * BLK_PER_W + jnp.minimum(wid, BLK_EXTRA)
    count = BLK_PER_W + extra
    last = count - 1

    def v0_of(g):
        return pl.multiple_of((start + jnp.minimum(g, last)) * TW, TW)

    def fire_load(g, b):
        pltpu.async_copy(tab_t.at[:, pl.ds(v0_of(g), TW)], src_v.at[b],
                         lsem[b])

    def wait_load(b):
        pltpu.make_async_copy(tab_t.at[:, pl.ds(0, TW)], src_v.at[b],
                              lsem[b]).wait()

    def fire_store(g, b):
        o = pl.multiple_of(v0_of(g) // 2, TW // 2)
        pltpu.async_copy(dst_v.at[b], out_hbm.at[pl.ds(o, TW // 2)], ssem[b])

    def wait_store(b):
        pltpu.make_async_copy(dst_v.at[b],
                              out_hbm.at[pl.ds(0, TW // 2)], ssem[b]).wait()

    def transpose_block(b):
        def body(u, _):
            for half in range(2):
                vidx = jnp.full((16,), 2 * u + half, jnp.int32)
                for d0 in range(4):
                    vals = plsc.load_gather(src_v.at[b],
                                            [d0 * 16 + iota, vidx])
                    dst_v[b, u, pl.ds(half * 64 + d0 * 16, 16)] = vals
            return 0

        lax.fori_loop(0, TW // 2, body, 0)

    fire_load(0, 0)
    fire_load(1, 1)
    for b in range(NBUF):
        wait_load(b)
        transpose_block(b)
        fire_store(b, b)
        fire_load(b + 2, b)

    npairs = (count - 2 + 1) // 2

    def blk_body(k, _):
        g0 = 2 + 2 * k
        for b in range(NBUF):
            g = g0 + b

            @pl.when(g < count)
            def _():
                wait_store(b)
                wait_load(b)
                transpose_block(b)
                fire_store(g, b)
                fire_load(g + 2, b)
        return 0

    lax.fori_loop(0, npairs, blk_body, 0)
    wait_store(0)
    wait_store(1)
    wait_load(0)
    wait_load(1)

    @pl.when(wid == NW - 1)
    def _():
        pltpu.sync_copy(tail_in, dst_v.at[0, pl.ds(0, 32)])
        pltpu.sync_copy(dst_v.at[0, pl.ds(0, 32)],
                        out_hbm.at[pl.ds(NBLK * (TW // 2), 32)])


@functools.partial(
    pl.kernel,
    mesh=_mesh,
    out_type=jax.ShapeDtypeStruct((ROWS128, 128, 128), jnp.float32),
    scratch_types=[
        pltpu.VMEM((NBUF, CHUNK_ROWS, 128), jnp.int32),
        pltpu.VMEM((NBUF, CHUNK_ROWS, 128, D), jnp.float32),
        pltpu.SemaphoreType.DMA,
        pltpu.SemaphoreType.DMA,
        pltpu.SemaphoreType.DMA,
        pltpu.SemaphoreType.DMA,
    ],
    compiler_params=pltpu.CompilerParams(use_tc_tiling_on_sc=False),
)
def _gather_kernel(tok_hbm, table_hbm, out_hbm, idx_v, rows_v,
                   gsem0, gsem1, ssem0, ssem1):
    wid = lax.axis_index("s") * 2 + lax.axis_index("c")
    base_row = wid * ROWS_PER_W
    gsem = (gsem0, gsem1)
    ssem = (ssem0, ssem1)

    def fire(g, b):
        row0 = base_row + g * CHUNK_ROWS
        pltpu.sync_copy(tok_hbm.at[pl.ds(row0, CHUNK_ROWS)], idx_v.at[b])
        for j in range(CHUNK_ROWS):
            pltpu.async_copy(table_hbm.at[idx_v.at[b, j]], rows_v.at[b, j],
                             gsem[b])

    def wait_gathers(b):
        pltpu.make_async_copy(
            out_hbm.at[pl.ds(0, CHUNK_ROWS), :, pl.ds(0, D)],
            rows_v.at[b], gsem[b]).wait()

    def fire_store(g, b):
        row0 = base_row + g * CHUNK_ROWS
        pltpu.async_copy(rows_v.at[b],
                         out_hbm.at[pl.ds(row0, CHUNK_ROWS), :, pl.ds(0, D)],
                         ssem[b])

    def wait_store(b):
        pltpu.make_async_copy(
            rows_v.at[b],
            out_hbm.at[pl.ds(0, CHUNK_ROWS), :, pl.ds(0, D)],
            ssem[b]).wait()

    fire(0, 0)
    fire(1, 1)
    wait_gathers(0)
    fire_store(0, 0)

    def body(k, _):
        g0 = 2 + 2 * k
        for b in range(NBUF):
            g = g0 + b
            wait_store(b)
            fire(g, b)
            wait_gathers(b ^ 1)
            fire_store(g - 1, b ^ 1)
        return 0

    lax.fori_loop(0, (NCHUNKS - 2) // 2, body, 0)

    b_last = (NCHUNKS - 1) % 2
    wait_gathers(b_last)
    fire_store(NCHUNKS - 1, b_last)
    wait_store(0)
    wait_store(1)


def kernel(tokens, word_embed_weight):
    tok = tokens.reshape(ROWS128, 128).astype(jnp.int32)
    tail_packed = word_embed_weight[NBLK * TW:].reshape(32, 128)
    packed = _transpose_kernel(word_embed_weight.T, tail_packed)
    table_rm = packed.reshape(VOCAB, D)
    out = _gather_kernel(tok, table_rm)
    return out.reshape(TOTAL, 128)[:, :D].reshape(B, L, D)


# speedup vs baseline: 1.4085x; 1.1819x over previous
import functools

import jax
import jax.numpy as jnp
from jax import lax
from jax.experimental import pallas as pl
from jax.experimental.pallas import tpu as pltpu
from jax.experimental.pallas import tpu_sc as plsc

B = 4096
L = 200
VOCAB = 1000000
D = 64

NW = 32
TOTAL = B * L
ROWS128 = TOTAL // 128
ROWS_PER_W = ROWS128 // NW
CHUNK_ROWS = 5
NCHUNKS = ROWS_PER_W // CHUNK_ROWS
NBUF = 2

TW = 256
NBLK = VOCAB // TW
BLK_PER_W = NBLK // NW
BLK_EXTRA = NBLK % NW

_mesh = plsc.VectorSubcoreMesh(core_axis_name="c", subcore_axis_name="s")


@functools.partial(
    pl.kernel,
    mesh=_mesh,
    out_type=jax.ShapeDtypeStruct((VOCAB // 2, 128), jnp.float32),
    scratch_types=[
        pltpu.VMEM((NBUF, D, TW), jnp.float32),
        pltpu.VMEM((NBUF, TW // 2, 128), jnp.float32),
        pltpu.SemaphoreType.DMA,
        pltpu.SemaphoreType.DMA,
        pltpu.SemaphoreType.DMA,
        pltpu.SemaphoreType.DMA,
    ],
    compiler_params=pltpu.CompilerParams(use_tc_tiling_on_sc=True,
                                         needs_layout_passes=False),
)
def _transpose_kernel(tab_t, tail_in, out_hbm, src_v, dst_v,
                      lsem0, lsem1, ssem0, ssem1):
    wid = lax.axis_index("s") * 2 + lax.axis_index("c")
    iota = lax.iota(jnp.int32, 16)
    lsem = (lsem0, lsem1)
    ssem = (ssem0, ssem1)

    extra = jnp.where(wid < BLK_EXTRA, 1, 0)
    start = wid * BLK_PER_W + jnp.minimum(wid, BLK_EXTRA)
    count = BLK_PER_W + extra
    last = count - 1

    def v0_of(g):
        return pl.multiple_of((start + jnp.minimum(g, last)) * TW, TW)

    def fire_load(g, b):
        pltpu.async_copy(tab_t.at[:, pl.ds(v0_of(g), TW)], src_v.at[b],
                         lsem[b])

    def wait_load(b):
        pltpu.make_async_copy(tab_t.at[:, pl.ds(0, TW)], src_v.at[b],
                              lsem[b]).wait()

    def fire_store(g, b):
        o = pl.multiple_of(v0_of(g) // 2, TW // 2)
        pltpu.async_copy(dst_v.at[b], out_hbm.at[pl.ds(o, TW // 2)], ssem[b])

    def wait_store(b):
        pltpu.make_async_copy(dst_v.at[b],
                              out_hbm.at[pl.ds(0, TW // 2)], ssem[b]).wait()

    rows_tab = [i * 8 + (iota >> 1) for i in range(TW // 16)]
    cols_half = (iota & 1) * 64

    def transpose_block(b):
        def body(d, _):
            cols = cols_half + d
            for i in range(TW // 16):
                vals = src_v[b, d, pl.ds(i * 16, 16)]
                plsc.store_scatter(dst_v.at[b], [rows_tab[i], cols], vals)
            return 0

        lax.fori_loop(0, D, body, 0)

    fire_load(0, 0)
    fire_load(1, 1)
    for b in range(NBUF):
        wait_load(b)
        transpose_block(b)
        fire_store(b, b)
        fire_load(b + 2, b)

    npairs = (count - 2 + 1) // 2

    def blk_body(k, _):
        g0 = 2 + 2 * k
        for b in range(NBUF):
            g = g0 + b

            @pl.when(g < count)
            def _():
                wait_store(b)
                wait_load(b)
                transpose_block(b)
                fire_store(g, b)
                fire_load(g + 2, b)
        return 0

    lax.fori_loop(0, npairs, blk_body, 0)
    wait_store(0)
    wait_store(1)
    wait_load(0)
    wait_load(1)

    @pl.when(wid == NW - 1)
    def _():
        pltpu.sync_copy(tail_in, dst_v.at[0, pl.ds(0, 32)])
        pltpu.sync_copy(dst_v.at[0, pl.ds(0, 32)],
                        out_hbm.at[pl.ds(NBLK * (TW // 2), 32)])


@functools.partial(
    pl.kernel,
    mesh=_mesh,
    out_type=jax.ShapeDtypeStruct((ROWS128, 128, 128), jnp.float32),
    scratch_types=[
        pltpu.VMEM((NBUF, CHUNK_ROWS, 128), jnp.int32),
        pltpu.VMEM((NBUF, CHUNK_ROWS, 128, D), jnp.float32),
        pltpu.SemaphoreType.DMA,
        pltpu.SemaphoreType.DMA,
        pltpu.SemaphoreType.DMA,
        pltpu.SemaphoreType.DMA,
    ],
    compiler_params=pltpu.CompilerParams(use_tc_tiling_on_sc=False),
)
def _gather_kernel(tok_hbm, table_hbm, out_hbm, idx_v, rows_v,
                   gsem0, gsem1, ssem0, ssem1):
    wid = lax.axis_index("s") * 2 + lax.axis_index("c")
    base_row = wid * ROWS_PER_W
    gsem = (gsem0, gsem1)
    ssem = (ssem0, ssem1)

    def fire(g, b):
        row0 = base_row + g * CHUNK_ROWS
        pltpu.sync_copy(tok_hbm.at[pl.ds(row0, CHUNK_ROWS)], idx_v.at[b])
        for j in range(CHUNK_ROWS):
            pltpu.async_copy(table_hbm.at[idx_v.at[b, j]], rows_v.at[b, j],
                             gsem[b])

    def wait_gathers(b):
        pltpu.make_async_copy(
            out_hbm.at[pl.ds(0, CHUNK_ROWS), :, pl.ds(0, D)],
            rows_v.at[b], gsem[b]).wait()

    def fire_store(g, b):
        row0 = base_row + g * CHUNK_ROWS
        pltpu.async_copy(rows_v.at[b],
                         out_hbm.at[pl.ds(row0, CHUNK_ROWS), :, pl.ds(0, D)],
                         ssem[b])

    def wait_store(b):
        pltpu.make_async_copy(
            rows_v.at[b],
            out_hbm.at[pl.ds(0, CHUNK_ROWS), :, pl.ds(0, D)],
            ssem[b]).wait()

    fire(0, 0)
    fire(1, 1)
    wait_gathers(0)
    fire_store(0, 0)

    def body(k, _):
        g0 = 2 + 2 * k
        for b in range(NBUF):
            g = g0 + b
            wait_store(b)
            fire(g, b)
            wait_gathers(b ^ 1)
            fire_store(g - 1, b ^ 1)
        return 0

    lax.fori_loop(0, (NCHUNKS - 2) // 2, body, 0)

    b_last = (NCHUNKS - 1) % 2
    wait_gathers(b_last)
    fire_store(NCHUNKS - 1, b_last)
    wait_store(0)
    wait_store(1)


def kernel(tokens, word_embed_weight):
    tok = tokens.reshape(ROWS128, 128).astype(jnp.int32)
    tail_packed = word_embed_weight[NBLK * TW:].reshape(32, 128)
    packed = _transpose_kernel(word_embed_weight.T, tail_packed)
    table_rm = packed.reshape(VOCAB, D)
    out = _gather_kernel(tok, table_rm)
    return out.reshape(TOTAL, 128)[:, :D].reshape(B, L, D)


# speedup vs baseline: 1.8182x; 1.2908x over previous
import functools

import jax
import jax.numpy as jnp
from jax import lax
from jax.experimental import pallas as pl
from jax.experimental.pallas import tpu as pltpu
from jax.experimental.pallas import tpu_sc as plsc

B = 4096
L = 200
VOCAB = 1000000
D = 64

NW = 32
TOTAL = B * L
ROWS128 = TOTAL // 128
ROWS_PER_W = ROWS128 // NW
CHUNK_ROWS = 5
NCHUNKS = ROWS_PER_W // CHUNK_ROWS
NBUF = 2

TW = 256
NBLK = VOCAB // TW
BLK_PER_W = NBLK // NW
BLK_EXTRA = NBLK % NW

_mesh = plsc.VectorSubcoreMesh(core_axis_name="c", subcore_axis_name="s")


@functools.partial(
    pl.kernel,
    mesh=_mesh,
    out_type=jax.ShapeDtypeStruct((VOCAB // 2, 128), jnp.float32),
    scratch_types=[
        pltpu.VMEM((NBUF, D, TW), jnp.float32),
        pltpu.VMEM((NBUF, TW // 2, 128), jnp.float32),
        pltpu.SemaphoreType.DMA,
        pltpu.SemaphoreType.DMA,
        pltpu.SemaphoreType.DMA,
        pltpu.SemaphoreType.DMA,
    ],
    compiler_params=pltpu.CompilerParams(use_tc_tiling_on_sc=True,
                                         needs_layout_passes=False),
)
def _transpose_kernel(tab_t, tail_in, out_hbm, src_v, dst_v,
                      lsem0, lsem1, ssem0, ssem1):
    wid = lax.axis_index("s") * 2 + lax.axis_index("c")
    iota = lax.iota(jnp.int32, 16)
    lsem = (lsem0, lsem1)
    ssem = (ssem0, ssem1)

    extra = jnp.where(wid < BLK_EXTRA, 1, 0)
    start = wid * BLK_PER_W + jnp.minimum(wid, BLK_EXTRA)
    count = BLK_PER_W + extra
    last = count - 1

    def v0_of(g):
        return pl.multiple_of((start + jnp.minimum(g, last)) * TW, TW)

    def fire_load(g, b):
        pltpu.async_copy(tab_t.at[:, pl.ds(v0_of(g), TW)], src_v.at[b],
                         lsem[b])

    def wait_load(b):
        pltpu.make_async_copy(tab_t.at[:, pl.ds(0, TW)], src_v.at[b],
                              lsem[b]).wait()

    def fire_store(g, b):
        o = pl.multiple_of(v0_of(g) // 2, TW // 2)
        pltpu.async_copy(dst_v.at[b], out_hbm.at[pl.ds(o, TW // 2)], ssem[b])

    def wait_store(b):
        pltpu.make_async_copy(dst_v.at[b],
                              out_hbm.at[pl.ds(0, TW // 2)], ssem[b]).wait()

    rows_tab = [i * 8 + (iota >> 1) for i in range(TW // 16)]
    cols_half = (iota & 1) * 64

    def transpose_block(b):
        @plsc.parallel_loop(0, D, unroll=4)
        def _(d):
            cols = cols_half + d
            for i in range(TW // 16):
                vals = src_v[b, d, pl.ds(i * 16, 16)]
                plsc.store_scatter(dst_v.at[b], [rows_tab[i], cols], vals)

    fire_load(0, 0)
    fire_load(1, 1)
    for b in range(NBUF):
        wait_load(b)
        transpose_block(b)
        fire_store(b, b)
        fire_load(b + 2, b)

    npairs = (count - 2 + 1) // 2

    def blk_body(k, _):
        g0 = 2 + 2 * k
        for b in range(NBUF):
            g = g0 + b

            @pl.when(g < count)
            def _():
                wait_store(b)
                wait_load(b)
                transpose_block(b)
                fire_store(g, b)
                fire_load(g + 2, b)
        return 0

    lax.fori_loop(0, npairs, blk_body, 0)
    wait_store(0)
    wait_store(1)
    wait_load(0)
    wait_load(1)

    @pl.when(wid == NW - 1)
    def _():
        pltpu.sync_copy(tail_in, dst_v.at[0, pl.ds(0, 32)])
        pltpu.sync_copy(dst_v.at[0, pl.ds(0, 32)],
                        out_hbm.at[pl.ds(NBLK * (TW // 2), 32)])


@functools.partial(
    pl.kernel,
    mesh=_mesh,
    out_type=jax.ShapeDtypeStruct((ROWS128, 128, 128), jnp.float32),
    scratch_types=[
        pltpu.VMEM((NBUF, CHUNK_ROWS, 128), jnp.int32),
        pltpu.VMEM((NBUF, CHUNK_ROWS, 128, D), jnp.float32),
        pltpu.SemaphoreType.DMA,
        pltpu.SemaphoreType.DMA,
        pltpu.SemaphoreType.DMA,
        pltpu.SemaphoreType.DMA,
    ],
    compiler_params=pltpu.CompilerParams(use_tc_tiling_on_sc=False),
)
def _gather_kernel(tok_hbm, table_hbm, out_hbm, idx_v, rows_v,
                   gsem0, gsem1, ssem0, ssem1):
    wid = lax.axis_index("s") * 2 + lax.axis_index("c")
    base_row = wid * ROWS_PER_W
    gsem = (gsem0, gsem1)
    ssem = (ssem0, ssem1)

    def fire(g, b):
        row0 = base_row + g * CHUNK_ROWS
        pltpu.sync_copy(tok_hbm.at[pl.ds(row0, CHUNK_ROWS)], idx_v.at[b])
        for j in range(CHUNK_ROWS):
            pltpu.async_copy(table_hbm.at[idx_v.at[b, j]], rows_v.at[b, j],
                             gsem[b])

    def wait_gathers(b):
        pltpu.make_async_copy(
            out_hbm.at[pl.ds(0, CHUNK_ROWS), :, pl.ds(0, D)],
            rows_v.at[b], gsem[b]).wait()

    def fire_store(g, b):
        row0 = base_row + g * CHUNK_ROWS
        pltpu.async_copy(rows_v.at[b],
                         out_hbm.at[pl.ds(row0, CHUNK_ROWS), :, pl.ds(0, D)],
                         ssem[b])

    def wait_store(b):
        pltpu.make_async_copy(
            rows_v.at[b],
            out_hbm.at[pl.ds(0, CHUNK_ROWS), :, pl.ds(0, D)],
            ssem[b]).wait()

    fire(0, 0)
    fire(1, 1)
    wait_gathers(0)
    fire_store(0, 0)

    def body(k, _):
        g0 = 2 + 2 * k
        for b in range(NBUF):
            g = g0 + b
            wait_store(b)
            fire(g, b)
            wait_gathers(b ^ 1)
            fire_store(g - 1, b ^ 1)
        return 0

    lax.fori_loop(0, (NCHUNKS - 2) // 2, body, 0)

    b_last = (NCHUNKS - 1) % 2
    wait_gathers(b_last)
    fire_store(NCHUNKS - 1, b_last)
    wait_store(0)
    wait_store(1)


def kernel(tokens, word_embed_weight):
    tok = tokens.reshape(ROWS128, 128).astype(jnp.int32)
    tail_packed = word_embed_weight[NBLK * TW:].reshape(32, 128)
    packed = _transpose_kernel(word_embed_weight.T, tail_packed)
    table_rm = packed.reshape(VOCAB, D)
    out = _gather_kernel(tok, table_rm)
    return out.reshape(TOTAL, 128)[:, :D].reshape(B, L, D)


# speedup vs baseline: 2.6700x; 1.4685x over previous
import functools

import jax
import jax.numpy as jnp
from jax import lax
from jax.experimental import pallas as pl
from jax.experimental.pallas import tpu as pltpu
from jax.experimental.pallas import tpu_sc as plsc

B = 4096
L = 200
VOCAB = 1000000
D = 64

NW = 32
TOTAL = B * L
ROWS128 = TOTAL // 128
ROWS_PER_W = ROWS128 // NW
CHUNK_ROWS = 5
NCHUNKS = ROWS_PER_W // CHUNK_ROWS
NBUF = 2

TW = 256
NBLK = VOCAB // TW
BLK_PER_W = NBLK // NW
BLK_EXTRA = NBLK % NW

_mesh = plsc.VectorSubcoreMesh(core_axis_name="c", subcore_axis_name="s")


@functools.partial(
    pl.kernel,
    mesh=_mesh,
    out_type=jax.ShapeDtypeStruct((VOCAB * D,), jnp.float32),
    scratch_types=[
        pltpu.VMEM((NBUF, D, TW), jnp.float32),
        pltpu.VMEM((TW * D,), jnp.float32),
        pltpu.VMEM((TW * D,), jnp.float32),
        pltpu.SemaphoreType.DMA,
        pltpu.SemaphoreType.DMA,
        pltpu.SemaphoreType.DMA,
        pltpu.SemaphoreType.DMA,
    ],
    compiler_params=pltpu.CompilerParams(use_tc_tiling_on_sc=True,
                                         needs_layout_passes=False),
)
def _transpose_kernel(tab_t, tail_in, out_hbm, src_v, dst_v0, dst_v1,
                      lsem0, lsem1, ssem0, ssem1):
    dst_v = (dst_v0, dst_v1)
    wid = lax.axis_index("s") * 2 + lax.axis_index("c")
    iota = lax.iota(jnp.int32, 16)
    lsem = (lsem0, lsem1)
    ssem = (ssem0, ssem1)

    extra = jnp.where(wid < BLK_EXTRA, 1, 0)
    start = wid * BLK_PER_W + jnp.minimum(wid, BLK_EXTRA)
    count = BLK_PER_W + extra
    last = count - 1

    def v0_of(g):
        return pl.multiple_of((start + jnp.minimum(g, last)) * TW, TW)

    def fire_load(g, b):
        pltpu.async_copy(tab_t.at[:, pl.ds(v0_of(g), TW)], src_v.at[b],
                         lsem[b])

    def wait_load(b):
        pltpu.make_async_copy(tab_t.at[:, pl.ds(0, TW)], src_v.at[b],
                              lsem[b]).wait()

    def fire_store(g, b):
        o = pl.multiple_of(v0_of(g) * D, TW * D)
        pltpu.async_copy(dst_v[b], out_hbm.at[pl.ds(o, TW * D)], ssem[b])

    def wait_store(b):
        pltpu.make_async_copy(dst_v[b],
                              out_hbm.at[pl.ds(0, TW * D)], ssem[b]).wait()

    didx_tab = [d0 + iota for d0 in range(0, D, 16)]
    vperm_tab = [(iota + k) % 16 for k in range(16)]
    wtab = [((iota + k) % 16) * D + iota for k in range(16)]

    def transpose_block(b):
        @plsc.parallel_loop(0, TW // 16, unroll=1)
        def _(vt):
            v0 = vt * 16
            for di, d0 in enumerate(range(0, D, 16)):
                base = v0 * D + d0
                for k in range(16):
                    vals = plsc.load_gather(src_v.at[b],
                                            [didx_tab[di], vperm_tab[k] + v0])
                    plsc.store_scatter(dst_v[b], [wtab[k] + base], vals)

    fire_load(0, 0)
    fire_load(1, 1)
    for b in range(NBUF):
        wait_load(b)
        transpose_block(b)
        fire_store(b, b)
        fire_load(b + 2, b)

    npairs = (count - 2 + 1) // 2

    def blk_body(k, _):
        g0 = 2 + 2 * k
        for b in range(NBUF):
            g = g0 + b

            @pl.when(g < count)
            def _():
                wait_store(b)
                wait_load(b)
                transpose_block(b)
                fire_store(g, b)
                fire_load(g + 2, b)
        return 0

    lax.fori_loop(0, npairs, blk_body, 0)
    wait_store(0)
    wait_store(1)
    wait_load(0)
    wait_load(1)

    @pl.when(wid == NW - 1)
    def _():
        pltpu.sync_copy(tail_in, dst_v0.at[pl.ds(0, 64 * D)])
        pltpu.sync_copy(dst_v0.at[pl.ds(0, 64 * D)],
                        out_hbm.at[pl.ds(NBLK * TW * D, 64 * D)])


@functools.partial(
    pl.kernel,
    mesh=_mesh,
    out_type=jax.ShapeDtypeStruct((ROWS128, 128, 128), jnp.float32),
    scratch_types=[
        pltpu.VMEM((NBUF, CHUNK_ROWS, 128), jnp.int32),
        pltpu.VMEM((NBUF, CHUNK_ROWS, 128, D), jnp.float32),
        pltpu.SemaphoreType.DMA,
        pltpu.SemaphoreType.DMA,
        pltpu.SemaphoreType.DMA,
        pltpu.SemaphoreType.DMA,
    ],
    compiler_params=pltpu.CompilerParams(use_tc_tiling_on_sc=False),
)
def _gather_kernel(tok_hbm, table_hbm, out_hbm, idx_v, rows_v,
                   gsem0, gsem1, ssem0, ssem1):
    wid = lax.axis_index("s") * 2 + lax.axis_index("c")
    base_row = wid * ROWS_PER_W
    gsem = (gsem0, gsem1)
    ssem = (ssem0, ssem1)

    def fire(g, b):
        row0 = base_row + g * CHUNK_ROWS
        pltpu.sync_copy(tok_hbm.at[pl.ds(row0, CHUNK_ROWS)], idx_v.at[b])
        for j in range(CHUNK_ROWS):
            pltpu.async_copy(table_hbm.at[idx_v.at[b, j]], rows_v.at[b, j],
                             gsem[b])

    def wait_gathers(b):
        pltpu.make_async_copy(
            out_hbm.at[pl.ds(0, CHUNK_ROWS), :, pl.ds(0, D)],
            rows_v.at[b], gsem[b]).wait()

    def fire_store(g, b):
        row0 = base_row + g * CHUNK_ROWS
        pltpu.async_copy(rows_v.at[b],
                         out_hbm.at[pl.ds(row0, CHUNK_ROWS), :, pl.ds(0, D)],
                         ssem[b])

    def wait_store(b):
        pltpu.make_async_copy(
            rows_v.at[b],
            out_hbm.at[pl.ds(0, CHUNK_ROWS), :, pl.ds(0, D)],
            ssem[b]).wait()

    fire(0, 0)
    fire(1, 1)
    wait_gathers(0)
    fire_store(0, 0)

    def body(k, _):
        g0 = 2 + 2 * k
        for b in range(NBUF):
            g = g0 + b
            wait_store(b)
            fire(g, b)
            wait_gathers(b ^ 1)
            fire_store(g - 1, b ^ 1)
        return 0

    lax.fori_loop(0, (NCHUNKS - 2) // 2, body, 0)

    b_last = (NCHUNKS - 1) % 2
    wait_gathers(b_last)
    fire_store(NCHUNKS - 1, b_last)
    wait_store(0)
    wait_store(1)


def kernel(tokens, word_embed_weight):
    tok = tokens.reshape(ROWS128, 128).astype(jnp.int32)
    tail_packed = word_embed_weight[NBLK * TW:].reshape(64 * D)
    packed = _transpose_kernel(word_embed_weight.T, tail_packed)
    table_rm = packed.reshape(VOCAB, D)
    out = _gather_kernel(tok, table_rm)
    return out.reshape(TOTAL, 128)[:, :D].reshape(B, L, D)
